# baseline (speedup 1.0000x reference)
import functools

import jax
import jax.numpy as jnp
from jax import lax
from jax.experimental import pallas as pl
from jax.experimental.pallas import tpu as pltpu
from jax.experimental.pallas import tpu_sc as plsc

_USER_NUM = 100000
_D = 64
_NF = 50
_B = 4096
_MEM = 8
_ATT = 16

_RB = 2048
_BB = 256
_NW = 32
_CH = 640


def _mlp_body(x_ref, iw_ref, wm_ref, bm_ref, wra_ref, wrb_ref, br_ref,
              o_ref, rn_ref):
    x = x_ref[...]
    s = jnp.maximum(
        jnp.dot(x, wm_ref[...], preferred_element_type=jnp.float32)
        + bm_ref[...][None, :], 0.0)
    y = (jnp.dot(x, wra_ref[...], preferred_element_type=jnp.float32)
         + jnp.dot(s, wrb_ref[...], preferred_element_type=jnp.float32)
         + br_ref[...][None, :])
    y = jnp.maximum(y, 0.0)
    o_ref[...] = jnp.concatenate([y, iw_ref[...]], axis=1)
    rn_ref[...] = 1.0 / jnp.maximum(
        jnp.sqrt(jnp.sum(y * y, axis=1)), 1e-12)


def _table_mlp(u2e, iidW, W_mlp, b_mlp, Wr_top, Wr_bot, b_r):
    n = u2e.shape[0]
    return pl.pallas_call(
        _mlp_body,
        grid=(pl.cdiv(n, _RB),),
        in_specs=[
            pl.BlockSpec((_RB, _D), lambda i: (i, 0)),
            pl.BlockSpec((_RB, _D), lambda i: (i, 0)),
            pl.BlockSpec((_D, _D), lambda i: (0, 0)),
            pl.BlockSpec((_D,), lambda i: (0,)),
            pl.BlockSpec((_D, _D), lambda i: (0, 0)),
            pl.BlockSpec((_D, _D), lambda i: (0, 0)),
            pl.BlockSpec((_D,), lambda i: (0,)),
        ],
        out_specs=[
            pl.BlockSpec((_RB, 2 * _D), lambda i: (i, 0)),
            pl.BlockSpec((_RB,), lambda i: (i,)),
        ],
        out_shape=[
            jax.ShapeDtypeStruct((n, 2 * _D), jnp.float32),
            jax.ShapeDtypeStruct((n,), jnp.float32),
        ],
    )(u2e, iidW, W_mlp, b_mlp, Wr_top, Wr_bot, b_r)


def _sc_gather(table, uf_flat, input_u, input_i):
    mesh = plsc.VectorSubcoreMesh(core_axis_name="c", subcore_axis_name="s")
    nchunk = (_NF * _B) // _NW // _CH
    pb = _B // _NW

    @functools.partial(
        pl.kernel,
        mesh=mesh,
        out_type=(
            jax.ShapeDtypeStruct((_NF * _B, 2 * _D), jnp.float32),
            jax.ShapeDtypeStruct((_B, 2 * _D), jnp.float32),
            jax.ShapeDtypeStruct((_B, 2 * _D), jnp.float32),
        ),
        scratch_types=[
            pltpu.VMEM((_CH,), jnp.int32),
            pltpu.VMEM((_CH, 2 * _D), jnp.float32),
            pltpu.VMEM((pb,), jnp.int32),
            pltpu.VMEM((pb, 2 * _D), jnp.float32),
            pltpu.SemaphoreType.DMA,
        ],
        compiler_params=pltpu.CompilerParams(use_tc_tiling_on_sc=True),
    )
    def gk(table_hbm, uf_hbm, u_hbm, i_hbm,
           fe_out, uid_out, iid_out,
           idx_v, rows_v, idx_s, rows_s, sem):
        wid = lax.axis_index("s") * 2 + lax.axis_index("c")
        fe_base = wid * (nchunk * _CH)

        def chunk(g, carry):
            off = fe_base + g * _CH
            pltpu.sync_copy(uf_hbm.at[pl.ds(off, _CH)], idx_v)
            cps = [
                pltpu.async_copy(table_hbm.at[idx_v.at[pl.ds(j * 128, 128)]],
                                 rows_v.at[pl.ds(j * 128, 128)], sem)
                for j in range(_CH // 128)
            ]
            for cp in cps:
                cp.wait()
            pltpu.sync_copy(rows_v, fe_out.at[pl.ds(off, _CH)])
            return carry

        lax.fori_loop(0, nchunk, chunk, 0)

        ub = wid * pb
        pltpu.sync_copy(u_hbm.at[pl.ds(ub, pb)], idx_s)
        pltpu.async_copy(table_hbm.at[idx_s], rows_s, sem).wait()
        pltpu.sync_copy(rows_s, uid_out.at[pl.ds(ub, pb)])

        pltpu.sync_copy(i_hbm.at[pl.ds(ub, pb)], idx_s)
        pltpu.async_copy(table_hbm.at[idx_s], rows_s, sem).wait()
        pltpu.sync_copy(rows_s, iid_out.at[pl.ds(ub, pb)])

    return gk(table, uf_flat, input_u, input_i)


def _sc_scalar_gather(i_bias, input_i, rnorm, uf_flat):
    mesh = plsc.VectorSubcoreMesh(core_axis_name="c", subcore_axis_name="s")
    pb = _B // _NW
    nchunk = (_NF * _B) // _NW // _CH

    @functools.partial(
        pl.kernel,
        mesh=mesh,
        out_type=(
            jax.ShapeDtypeStruct((_B,), jnp.float32),
            jax.ShapeDtypeStruct((_NF * _B,), jnp.float32),
        ),
        scratch_types=[
            pltpu.VMEM((pb,), jnp.int32),
            pltpu.VMEM((pb,), jnp.float32),
            pltpu.VMEM((_CH,), jnp.int32),
            pltpu.VMEM((_CH,), jnp.float32),
            pltpu.SemaphoreType.DMA,
        ],
        compiler_params=pltpu.CompilerParams(use_tc_tiling_on_sc=False),
    )
    def bk(ibias_hbm, i_hbm, rn_hbm, uf_hbm, ib_out, rng_out,
           idx_s, ib_v, idx_v, rn_v, sem):
        wid = lax.axis_index("s") * 2 + lax.axis_index("c")
        ub = wid * pb
        pltpu.sync_copy(i_hbm.at[pl.ds(ub, pb)], idx_s)
        pltpu.async_copy(ibias_hbm.at[idx_s], ib_v, sem).wait()
        pltpu.sync_copy(ib_v, ib_out.at[pl.ds(ub, pb)])

        base = wid * (nchunk * _CH)

        def chunk(g, carry):
            off = base + g * _CH
            pltpu.sync_copy(uf_hbm.at[pl.ds(off, _CH)], idx_v)
            cps = [
                pltpu.async_copy(rn_hbm.at[idx_v.at[pl.ds(k * 128, 128)]],
                                 rn_v.at[pl.ds(k * 128, 128)], sem)
                for k in range(_CH // 128)
            ]
            for cp in cps:
                cp.wait()
            pltpu.sync_copy(rn_v, rng_out.at[pl.ds(off, _CH)])
            return carry

        lax.fori_loop(0, nchunk, chunk, 0)

    return bk(i_bias, input_i, rnorm, uf_flat)


_NBLK = _B // _BB


def _att_body(fe_ref, uf_ref, rng_ref, uid_ref, iid_ref, ib_ref,
              key_ref, mem_ref, wa_ref, ba_ref, uo_ref,
              score_ref, eak_ref, den_ref):
    i = pl.program_id(0)
    j = lax.rem(i, _NBLK)


    @pl.when(i < _NBLK)
    def _pass1():
        fe = fe_ref[:, :, :_D]
        uid = uid_ref[:, :_D]
        un = uid / jnp.maximum(
            jnp.sqrt(jnp.sum(uid * uid, axis=1, keepdims=True)), 1e-12)
        cross = un[None, :, :] * fe
        ak = jnp.dot(cross.reshape(_NF * _BB, _D), key_ref[...],
                     preferred_element_type=jnp.float32)
        akt = jnp.swapaxes(ak.reshape(_NF, _BB, _MEM), 1, 2)
        akt = akt * rng_ref[...][:, None, :]
        fmask = uf_ref[...] != _USER_NUM
        eakt = jnp.where(fmask[:, None, :], jnp.exp(akt), 1.0)
        eak_ref[:, :, pl.ds(j * _BB, _BB)] = eakt
        part = jnp.sum(eakt, axis=2)

        @pl.when(i == 0)
        def _():
            den_ref[...] = jnp.zeros_like(den_ref)

        den_ref[...] += part

    @pl.when(i >= _NBLK)
    def _pass2():
        fmask = uf_ref[...] != _USER_NUM
        fe = fe_ref[:, :, :_D]
        uid = uid_ref[:, :_D]
        amt = jnp.where(
            fmask[:, None, :],
            eak_ref[:, :, pl.ds(j * _BB, _BB)] / den_ref[...][:, :, None],
            0.0)
        am = jnp.swapaxes(amt, 1, 2)
        f1 = jnp.dot(am.reshape(_NF * _BB, _MEM), mem_ref[...],
                     preferred_element_type=jnp.float32)
        f2 = f1.reshape(_NF, _BB, _D) * fe
        h = jnp.maximum(
            jnp.dot(f2.reshape(_NF * _BB, _D), wa_ref[...],
                    preferred_element_type=jnp.float32)
            + ba_ref[...][None, :], 0.0)
        fj = jnp.exp(jnp.sum(h.reshape(_NF, _BB, _ATT)
                             * uo_ref[...][None, None, :], axis=2))
        fj = jnp.where(fmask, fj, 0.0)
        fsum = jnp.sum(fj, axis=0) + 1e-8
        fw = fj / fsum[None, :]
        friend = jnp.sum(fw[:, :, None] * f2, axis=0)
        user = uid + friend
        score_ref[...] = (jnp.sum(user * iid_ref[:, _D:], axis=1)
                          + ib_ref[...])


def _attention(fe3, uf_t, rng2, uid, iid, ib, Key, Mem, WA, BA, uo):
    return pl.pallas_call(
        _att_body,
        grid=(2 * _NBLK,),
        in_specs=[
            pl.BlockSpec((_NF, _BB, 2 * _D),
                         lambda i: (0, lax.rem(i, _NBLK), 0)),
            pl.BlockSpec((_NF, _BB), lambda i: (0, lax.rem(i, _NBLK))),
            pl.BlockSpec((_NF, _BB), lambda i: (0, lax.rem(i, _NBLK))),
            pl.BlockSpec((_BB, 2 * _D), lambda i: (lax.rem(i, _NBLK), 0)),
            pl.BlockSpec((_BB, 2 * _D), lambda i: (lax.rem(i, _NBLK), 0)),
            pl.BlockSpec((_BB,), lambda i: (lax.rem(i, _NBLK),)),
            pl.BlockSpec((_D, _MEM), lambda i: (0, 0)),
            pl.BlockSpec((_MEM, _D), lambda i: (0, 0)),
            pl.BlockSpec((_D, _ATT), lambda i: (0, 0)),
            pl.BlockSpec((_ATT,), lambda i: (0,)),
            pl.BlockSpec((_ATT,), lambda i: (0,)),
        ],
        out_specs=pl.BlockSpec((_BB,), lambda i: (lax.rem(i, _NBLK),)),
        out_shape=jax.ShapeDtypeStruct((_B,), jnp.float32),
        scratch_shapes=[
            pltpu.VMEM((_NF, _MEM, _B), jnp.float32),
            pltpu.VMEM((_NF, _MEM), jnp.float32),
        ],
    )(fe3, uf_t, rng2, uid, iid, ib, Key, Mem, WA, BA, uo)


def kernel(input_u, input_i, label, input_uf, i, data, flag,
           u2e_r, u2e_t, iidW, i_bias, W_mlp, b_mlp, W_r, b_r, W_t, b_t,
           Key, Mem, WA, BA, U_omega):
    input_u = input_u.astype(jnp.int32)
    input_i = input_i.astype(jnp.int32)
    uf_t = input_uf.astype(jnp.int32).T

    uf_flat = uf_t.reshape(-1)
    table, rnorm = _table_mlp(u2e_r, iidW, W_mlp, b_mlp,
                              W_r[:_D], W_r[_D:], b_r)
    fe_flat, uid, iid = _sc_gather(table, uf_flat, input_u, input_i)
    ib, rng = _sc_scalar_gather(i_bias, input_i, rnorm, uf_flat)
    fe3 = fe_flat.reshape(_NF, _B, 2 * _D)
    return _attention(fe3, uf_t, rng.reshape(_NF, _B), uid, iid, ib,
                      Key, Mem, WA, BA, U_omega[:, 0])


# speedup vs baseline: 1.1820x; 1.1820x over previous
import functools

import jax
import jax.numpy as jnp
from jax import lax
from jax.experimental import pallas as pl
from jax.experimental.pallas import tpu as pltpu
from jax.experimental.pallas import tpu_sc as plsc

_USER_NUM = 100000
_D = 64
_NF = 50
_B = 4096
_MEM = 8
_ATT = 16

_RB = 2048
_BB = 256
_NW = 32
_CH = 640


def _mlp_body(x_ref, iw_ref, wm_ref, bm_ref, wra_ref, wrb_ref, br_ref, o_ref):
    x = x_ref[...]
    s = jnp.maximum(
        jnp.dot(x, wm_ref[...], preferred_element_type=jnp.float32)
        + bm_ref[...][None, :], 0.0)
    y = (jnp.dot(x, wra_ref[...], preferred_element_type=jnp.float32)
         + jnp.dot(s, wrb_ref[...], preferred_element_type=jnp.float32)
         + br_ref[...][None, :])
    o_ref[...] = jnp.concatenate(
        [jnp.maximum(y, 0.0), iw_ref[...]], axis=1)


def _table_mlp(u2e, iidW, W_mlp, b_mlp, Wr_top, Wr_bot, b_r):
    n = u2e.shape[0]
    return pl.pallas_call(
        _mlp_body,
        grid=(pl.cdiv(n, _RB),),
        in_specs=[
            pl.BlockSpec((_RB, _D), lambda i: (i, 0)),
            pl.BlockSpec((_RB, _D), lambda i: (i, 0)),
            pl.BlockSpec((_D, _D), lambda i: (0, 0)),
            pl.BlockSpec((_D,), lambda i: (0,)),
            pl.BlockSpec((_D, _D), lambda i: (0, 0)),
            pl.BlockSpec((_D, _D), lambda i: (0, 0)),
            pl.BlockSpec((_D,), lambda i: (0,)),
        ],
        out_specs=pl.BlockSpec((_RB, 2 * _D), lambda i: (i, 0)),
        out_shape=jax.ShapeDtypeStruct((n, 2 * _D), jnp.float32),
    )(u2e, iidW, W_mlp, b_mlp, Wr_top, Wr_bot, b_r)


def _sc_gather(table, uf_flat, input_u, input_i):
    mesh = plsc.VectorSubcoreMesh(core_axis_name="c", subcore_axis_name="s")
    nchunk = (_NF * _B) // _NW // _CH
    pb = _B // _NW

    @functools.partial(
        pl.kernel,
        mesh=mesh,
        out_type=(
            jax.ShapeDtypeStruct((_NF * _B, 2 * _D), jnp.float32),
            jax.ShapeDtypeStruct((_B, 2 * _D), jnp.float32),
            jax.ShapeDtypeStruct((_B, 2 * _D), jnp.float32),
        ),
        scratch_types=[
            pltpu.VMEM((_CH,), jnp.int32),
            pltpu.VMEM((_CH, 2 * _D), jnp.float32),
            pltpu.VMEM((pb,), jnp.int32),
            pltpu.VMEM((pb, 2 * _D), jnp.float32),
            pltpu.SemaphoreType.DMA,
        ],
        compiler_params=pltpu.CompilerParams(use_tc_tiling_on_sc=True),
    )
    def gk(table_hbm, uf_hbm, u_hbm, i_hbm,
           fe_out, uid_out, iid_out,
           idx_v, rows_v, idx_s, rows_s, sem):
        wid = lax.axis_index("s") * 2 + lax.axis_index("c")
        fe_base = wid * (nchunk * _CH)

        def chunk(g, carry):
            off = fe_base + g * _CH
            pltpu.sync_copy(uf_hbm.at[pl.ds(off, _CH)], idx_v)
            cps = [
                pltpu.async_copy(table_hbm.at[idx_v.at[pl.ds(j * 128, 128)]],
                                 rows_v.at[pl.ds(j * 128, 128)], sem)
                for j in range(_CH // 128)
            ]
            for cp in cps:
                cp.wait()
            pltpu.sync_copy(rows_v, fe_out.at[pl.ds(off, _CH)])
            return carry

        lax.fori_loop(0, nchunk, chunk, 0)

        ub = wid * pb
        pltpu.sync_copy(u_hbm.at[pl.ds(ub, pb)], idx_s)
        pltpu.async_copy(table_hbm.at[idx_s], rows_s, sem).wait()
        pltpu.sync_copy(rows_s, uid_out.at[pl.ds(ub, pb)])

        pltpu.sync_copy(i_hbm.at[pl.ds(ub, pb)], idx_s)
        pltpu.async_copy(table_hbm.at[idx_s], rows_s, sem).wait()
        pltpu.sync_copy(rows_s, iid_out.at[pl.ds(ub, pb)])

    return gk(table, uf_flat, input_u, input_i)


def _sc_bias_gather(i_bias, input_i):
    mesh = plsc.VectorSubcoreMesh(core_axis_name="c", subcore_axis_name="s")
    pb = _B // _NW

    @functools.partial(
        pl.kernel,
        mesh=mesh,
        out_type=jax.ShapeDtypeStruct((_B,), jnp.float32),
        scratch_types=[
            pltpu.VMEM((pb,), jnp.int32),
            pltpu.VMEM((pb,), jnp.float32),
            pltpu.SemaphoreType.DMA,
        ],
        compiler_params=pltpu.CompilerParams(use_tc_tiling_on_sc=False),
    )
    def bk(ibias_hbm, i_hbm, ib_out, idx_s, ib_v, sem):
        wid = lax.axis_index("s") * 2 + lax.axis_index("c")
        ub = wid * pb
        pltpu.sync_copy(i_hbm.at[pl.ds(ub, pb)], idx_s)
        pltpu.async_copy(ibias_hbm.at[idx_s], ib_v, sem).wait()
        pltpu.sync_copy(ib_v, ib_out.at[pl.ds(ub, pb)])

    return bk(i_bias, input_i)


_NBLK = _B // _BB


def _att_body(fe_ref, uf_ref, uid_ref, iid_ref, ib_ref,
              key_ref, mem_ref, wa_ref, ba_ref, uo_ref,
              score_ref, eak_ref, den_ref):
    i = pl.program_id(0)
    j = lax.rem(i, _NBLK)


    @pl.when(i < _NBLK)
    def _pass1():
        fe = fe_ref[:, :, :_D]
        uid = uid_ref[:, :_D]
        un = uid / jnp.maximum(
            jnp.sqrt(jnp.sum(uid * uid, axis=1, keepdims=True)), 1e-12)
        fen = fe / jnp.maximum(
            jnp.sqrt(jnp.sum(fe * fe, axis=2, keepdims=True)), 1e-12)
        cross = un[None, :, :] * fen
        ak = jnp.dot(cross.reshape(_NF * _BB, _D), key_ref[...],
                     preferred_element_type=jnp.float32)
        akt = jnp.swapaxes(ak.reshape(_NF, _BB, _MEM), 1, 2)
        fmask = uf_ref[...] != _USER_NUM
        eakt = jnp.where(fmask[:, None, :], jnp.exp(akt), 1.0)
        eak_ref[:, :, pl.ds(j * _BB, _BB)] = eakt
        part = jnp.sum(eakt, axis=2)

        @pl.when(i == 0)
        def _():
            den_ref[...] = jnp.zeros_like(den_ref)

        den_ref[...] += part

    @pl.when(i >= _NBLK)
    def _pass2():
        fmask = uf_ref[...] != _USER_NUM
        fe = fe_ref[:, :, :_D]
        uid = uid_ref[:, :_D]
        amt = jnp.where(
            fmask[:, None, :],
            eak_ref[:, :, pl.ds(j * _BB, _BB)] / den_ref[...][:, :, None],
            0.0)
        am = jnp.swapaxes(amt, 1, 2)
        f1 = jnp.dot(am.reshape(_NF * _BB, _MEM), mem_ref[...],
                     preferred_element_type=jnp.float32)
        f2 = f1.reshape(_NF, _BB, _D) * fe
        h = jnp.maximum(
            jnp.dot(f2.reshape(_NF * _BB, _D), wa_ref[...],
                    preferred_element_type=jnp.float32)
            + ba_ref[...][None, :], 0.0)
        fj = jnp.exp(jnp.sum(h.reshape(_NF, _BB, _ATT)
                             * uo_ref[...][None, None, :], axis=2))
        fj = jnp.where(fmask, fj, 0.0)
        fsum = jnp.sum(fj, axis=0) + 1e-8
        fw = fj / fsum[None, :]
        friend = jnp.sum(fw[:, :, None] * f2, axis=0)
        user = uid + friend
        score_ref[...] = (jnp.sum(user * iid_ref[:, _D:], axis=1)
                          + ib_ref[...])


def _attention(fe3, uf_t, uid, iid, ib, Key, Mem, WA, BA, uo):
    return pl.pallas_call(
        _att_body,
        grid=(2 * _NBLK,),
        in_specs=[
            pl.BlockSpec((_NF, _BB, 2 * _D),
                         lambda i: (0, lax.rem(i, _NBLK), 0)),
            pl.BlockSpec((_NF, _BB), lambda i: (0, lax.rem(i, _NBLK))),
            pl.BlockSpec((_BB, 2 * _D), lambda i: (lax.rem(i, _NBLK), 0)),
            pl.BlockSpec((_BB, 2 * _D), lambda i: (lax.rem(i, _NBLK), 0)),
            pl.BlockSpec((_BB,), lambda i: (lax.rem(i, _NBLK),)),
            pl.BlockSpec((_D, _MEM), lambda i: (0, 0)),
            pl.BlockSpec((_MEM, _D), lambda i: (0, 0)),
            pl.BlockSpec((_D, _ATT), lambda i: (0, 0)),
            pl.BlockSpec((_ATT,), lambda i: (0,)),
            pl.BlockSpec((_ATT,), lambda i: (0,)),
        ],
        out_specs=pl.BlockSpec((_BB,), lambda i: (lax.rem(i, _NBLK),)),
        out_shape=jax.ShapeDtypeStruct((_B,), jnp.float32),
        scratch_shapes=[
            pltpu.VMEM((_NF, _MEM, _B), jnp.float32),
            pltpu.VMEM((_NF, _MEM), jnp.float32),
        ],
    )(fe3, uf_t, uid, iid, ib, Key, Mem, WA, BA, uo)


def kernel(input_u, input_i, label, input_uf, i, data, flag,
           u2e_r, u2e_t, iidW, i_bias, W_mlp, b_mlp, W_r, b_r, W_t, b_t,
           Key, Mem, WA, BA, U_omega):
    input_u = input_u.astype(jnp.int32)
    input_i = input_i.astype(jnp.int32)
    uf_t = input_uf.astype(jnp.int32).T

    table = _table_mlp(u2e_r, iidW, W_mlp, b_mlp, W_r[:_D], W_r[_D:], b_r)
    fe_flat, uid, iid = _sc_gather(table, uf_t.reshape(-1), input_u, input_i)
    ib = _sc_bias_gather(i_bias, input_i)
    fe3 = fe_flat.reshape(_NF, _B, 2 * _D)
    return _attention(fe3, uf_t, uid, iid, ib,
                      Key, Mem, WA, BA, U_omega[:, 0])


# speedup vs baseline: 1.1836x; 1.0014x over previous
import functools

import jax
import jax.numpy as jnp
from jax import lax
from jax.experimental import pallas as pl
from jax.experimental.pallas import tpu as pltpu
from jax.experimental.pallas import tpu_sc as plsc

_USER_NUM = 100000
_D = 64
_NF = 50
_B = 4096
_MEM = 8
_ATT = 16

_RB = 2048
_BB = 512
_NW = 32
_CH = 640


def _mlp_body(x_ref, iw_ref, wm_ref, bm_ref, wra_ref, wrb_ref, br_ref, o_ref):
    x = x_ref[...]
    s = jnp.maximum(
        jnp.dot(x, wm_ref[...], preferred_element_type=jnp.float32)
        + bm_ref[...][None, :], 0.0)
    y = (jnp.dot(x, wra_ref[...], preferred_element_type=jnp.float32)
         + jnp.dot(s, wrb_ref[...], preferred_element_type=jnp.float32)
         + br_ref[...][None, :])
    o_ref[...] = jnp.concatenate(
        [jnp.maximum(y, 0.0), iw_ref[...]], axis=1)


def _table_mlp(u2e, iidW, W_mlp, b_mlp, Wr_top, Wr_bot, b_r):
    n = u2e.shape[0]
    return pl.pallas_call(
        _mlp_body,
        grid=(pl.cdiv(n, _RB),),
        in_specs=[
            pl.BlockSpec((_RB, _D), lambda i: (i, 0)),
            pl.BlockSpec((_RB, _D), lambda i: (i, 0)),
            pl.BlockSpec((_D, _D), lambda i: (0, 0)),
            pl.BlockSpec((_D,), lambda i: (0,)),
            pl.BlockSpec((_D, _D), lambda i: (0, 0)),
            pl.BlockSpec((_D, _D), lambda i: (0, 0)),
            pl.BlockSpec((_D,), lambda i: (0,)),
        ],
        out_specs=pl.BlockSpec((_RB, 2 * _D), lambda i: (i, 0)),
        out_shape=jax.ShapeDtypeStruct((n, 2 * _D), jnp.float32),
    )(u2e, iidW, W_mlp, b_mlp, Wr_top, Wr_bot, b_r)


def _sc_gather(table, uf_flat, input_u, input_i):
    mesh = plsc.VectorSubcoreMesh(core_axis_name="c", subcore_axis_name="s")
    nchunk = (_NF * _B) // _NW // _CH
    pb = _B // _NW

    @functools.partial(
        pl.kernel,
        mesh=mesh,
        out_type=(
            jax.ShapeDtypeStruct((_NF * _B, 2 * _D), jnp.float32),
            jax.ShapeDtypeStruct((_B, 2 * _D), jnp.float32),
            jax.ShapeDtypeStruct((_B, 2 * _D), jnp.float32),
        ),
        scratch_types=[
            pltpu.VMEM((_CH,), jnp.int32),
            pltpu.VMEM((_CH, 2 * _D), jnp.float32),
            pltpu.VMEM((pb,), jnp.int32),
            pltpu.VMEM((pb, 2 * _D), jnp.float32),
            pltpu.SemaphoreType.DMA,
        ],
        compiler_params=pltpu.CompilerParams(use_tc_tiling_on_sc=True),
    )
    def gk(table_hbm, uf_hbm, u_hbm, i_hbm,
           fe_out, uid_out, iid_out,
           idx_v, rows_v, idx_s, rows_s, sem):
        wid = lax.axis_index("s") * 2 + lax.axis_index("c")
        fe_base = wid * (nchunk * _CH)

        def chunk(g, carry):
            off = fe_base + g * _CH
            pltpu.sync_copy(uf_hbm.at[pl.ds(off, _CH)], idx_v)
            cps = [
                pltpu.async_copy(table_hbm.at[idx_v.at[pl.ds(j * 128, 128)]],
                                 rows_v.at[pl.ds(j * 128, 128)], sem)
                for j in range(_CH // 128)
            ]
            for cp in cps:
                cp.wait()
            pltpu.sync_copy(rows_v, fe_out.at[pl.ds(off, _CH)])
            return carry

        lax.fori_loop(0, nchunk, chunk, 0)

        ub = wid * pb
        pltpu.sync_copy(u_hbm.at[pl.ds(ub, pb)], idx_s)
        pltpu.async_copy(table_hbm.at[idx_s], rows_s, sem).wait()
        pltpu.sync_copy(rows_s, uid_out.at[pl.ds(ub, pb)])

        pltpu.sync_copy(i_hbm.at[pl.ds(ub, pb)], idx_s)
        pltpu.async_copy(table_hbm.at[idx_s], rows_s, sem).wait()
        pltpu.sync_copy(rows_s, iid_out.at[pl.ds(ub, pb)])

    return gk(table, uf_flat, input_u, input_i)


def _sc_bias_gather(i_bias, input_i):
    mesh = plsc.VectorSubcoreMesh(core_axis_name="c", subcore_axis_name="s")
    pb = _B // _NW

    @functools.partial(
        pl.kernel,
        mesh=mesh,
        out_type=jax.ShapeDtypeStruct((_B,), jnp.float32),
        scratch_types=[
            pltpu.VMEM((pb,), jnp.int32),
            pltpu.VMEM((pb,), jnp.float32),
            pltpu.SemaphoreType.DMA,
        ],
        compiler_params=pltpu.CompilerParams(use_tc_tiling_on_sc=False),
    )
    def bk(ibias_hbm, i_hbm, ib_out, idx_s, ib_v, sem):
        wid = lax.axis_index("s") * 2 + lax.axis_index("c")
        ub = wid * pb
        pltpu.sync_copy(i_hbm.at[pl.ds(ub, pb)], idx_s)
        pltpu.async_copy(ibias_hbm.at[idx_s], ib_v, sem).wait()
        pltpu.sync_copy(ib_v, ib_out.at[pl.ds(ub, pb)])

    return bk(i_bias, input_i)


_NBLK = _B // _BB


def _att_body(fe_ref, uf_ref, uid_ref, iid_ref, ib_ref,
              key_ref, mem_ref, wa_ref, ba_ref, uo_ref,
              score_ref, eak_ref, den_ref):
    i = pl.program_id(0)
    j = lax.rem(i, _NBLK)


    @pl.when(i < _NBLK)
    def _pass1():
        fe = fe_ref[:, :, :_D]
        uid = uid_ref[:, :_D]
        un = uid / jnp.maximum(
            jnp.sqrt(jnp.sum(uid * uid, axis=1, keepdims=True)), 1e-12)
        fen = fe / jnp.maximum(
            jnp.sqrt(jnp.sum(fe * fe, axis=2, keepdims=True)), 1e-12)
        cross = un[None, :, :] * fen
        ak = jnp.dot(cross.reshape(_NF * _BB, _D), key_ref[...],
                     preferred_element_type=jnp.float32)
        akt = jnp.swapaxes(ak.reshape(_NF, _BB, _MEM), 1, 2)
        fmask = uf_ref[...] != _USER_NUM
        eakt = jnp.where(fmask[:, None, :], jnp.exp(akt), 1.0)
        eak_ref[:, :, pl.ds(j * _BB, _BB)] = eakt
        part = jnp.sum(eakt, axis=2)

        @pl.when(i == 0)
        def _():
            den_ref[...] = jnp.zeros_like(den_ref)

        den_ref[...] += part

    @pl.when(i >= _NBLK)
    def _pass2():
        fmask = uf_ref[...] != _USER_NUM
        fe = fe_ref[:, :, :_D]
        uid = uid_ref[:, :_D]
        amt = jnp.where(
            fmask[:, None, :],
            eak_ref[:, :, pl.ds(j * _BB, _BB)] / den_ref[...][:, :, None],
            0.0)
        am = jnp.swapaxes(amt, 1, 2)
        f1 = jnp.dot(am.reshape(_NF * _BB, _MEM), mem_ref[...],
                     preferred_element_type=jnp.float32)
        f2 = f1.reshape(_NF, _BB, _D) * fe
        h = jnp.maximum(
            jnp.dot(f2.reshape(_NF * _BB, _D), wa_ref[...],
                    preferred_element_type=jnp.float32)
            + ba_ref[...][None, :], 0.0)
        fj = jnp.exp(jnp.sum(h.reshape(_NF, _BB, _ATT)
                             * uo_ref[...][None, None, :], axis=2))
        fj = jnp.where(fmask, fj, 0.0)
        fsum = jnp.sum(fj, axis=0) + 1e-8
        fw = fj / fsum[None, :]
        friend = jnp.sum(fw[:, :, None] * f2, axis=0)
        user = uid + friend
        score_ref[...] = (jnp.sum(user * iid_ref[:, _D:], axis=1)
                          + ib_ref[...])


def _attention(fe3, uf_t, uid, iid, ib, Key, Mem, WA, BA, uo):
    return pl.pallas_call(
        _att_body,
        grid=(2 * _NBLK,),
        in_specs=[
            pl.BlockSpec((_NF, _BB, 2 * _D),
                         lambda i: (0, lax.rem(i, _NBLK), 0)),
            pl.BlockSpec((_NF, _BB), lambda i: (0, lax.rem(i, _NBLK))),
            pl.BlockSpec((_BB, 2 * _D), lambda i: (lax.rem(i, _NBLK), 0)),
            pl.BlockSpec((_BB, 2 * _D), lambda i: (lax.rem(i, _NBLK), 0)),
            pl.BlockSpec((_BB,), lambda i: (lax.rem(i, _NBLK),)),
            pl.BlockSpec((_D, _MEM), lambda i: (0, 0)),
            pl.BlockSpec((_MEM, _D), lambda i: (0, 0)),
            pl.BlockSpec((_D, _ATT), lambda i: (0, 0)),
            pl.BlockSpec((_ATT,), lambda i: (0,)),
            pl.BlockSpec((_ATT,), lambda i: (0,)),
        ],
        out_specs=pl.BlockSpec((_BB,), lambda i: (lax.rem(i, _NBLK),)),
        out_shape=jax.ShapeDtypeStruct((_B,), jnp.float32),
        scratch_shapes=[
            pltpu.VMEM((_NF, _MEM, _B), jnp.float32),
            pltpu.VMEM((_NF, _MEM), jnp.float32),
        ],
    )(fe3, uf_t, uid, iid, ib, Key, Mem, WA, BA, uo)


def kernel(input_u, input_i, label, input_uf, i, data, flag,
           u2e_r, u2e_t, iidW, i_bias, W_mlp, b_mlp, W_r, b_r, W_t, b_t,
           Key, Mem, WA, BA, U_omega):
    input_u = input_u.astype(jnp.int32)
    input_i = input_i.astype(jnp.int32)
    uf_t = input_uf.astype(jnp.int32).T

    table = _table_mlp(u2e_r, iidW, W_mlp, b_mlp, W_r[:_D], W_r[_D:], b_r)
    fe_flat, uid, iid = _sc_gather(table, uf_t.reshape(-1), input_u, input_i)
    ib = _sc_bias_gather(i_bias, input_i)
    fe3 = fe_flat.reshape(_NF, _B, 2 * _D)
    return _attention(fe3, uf_t, uid, iid, ib,
                      Key, Mem, WA, BA, U_omega[:, 0])
